# baseline (device time: 15953 ns/iter reference)
import jax
import jax.numpy as jnp
from jax import lax
from jax.experimental import pallas as pl
from jax.experimental.pallas import tpu as pltpu

N_DEV = 4
B = 2
SQ = 128
D_MODEL = 512
HQ = 4
DH = 64
SKV_LOC = 128
BLK = 64
SCALE = 0.125


def kernel(x, Wq, K_ext, V_ext, Wo):
    Kt = jnp.transpose(K_ext, (0, 2, 3, 1))
    Vt = jnp.transpose(V_ext, (0, 2, 3, 1))

    def body(x_hbm, wq_hbm, kt_hbm, vt_hbm, wo_hbm, out_ref,
             x_v, wq_v, wo_v, kt_v, vt_v, kv16,
             load_sems, send_sems, recv_sems):
        my = lax.axis_index("i")

        def load(hbm, vmem, i):
            return pltpu.make_async_copy(hbm, vmem, load_sems.at[i])

        def kv_copy(i, t):
            return pltpu.make_async_remote_copy(
                src_ref=kv16.at[i],
                dst_ref=kv16.at[i],
                send_sem=send_sems.at[i * (N_DEV - 1) + max(t - 1, 0)],
                recv_sem=recv_sems.at[i],
                device_id=(t,),
                device_id_type=pl.DeviceIdType.MESH,
            )

        load(x_hbm, x_v, 0).start()
        load(wq_hbm, wq_v, 1).start()
        load(wo_hbm, wo_v, 2).start()

        barrier = pltpu.get_barrier_semaphore()

        @pl.when(my != 0)
        def _():
            pl.semaphore_signal(
                barrier, inc=1, device_id=(0,),
                device_id_type=pl.DeviceIdType.MESH,
            )

        @pl.when(my == 0)
        def _():
            load(kt_hbm, kt_v, 3).start()
            load(vt_hbm, vt_v, 4).start()
            pl.semaphore_wait(barrier, N_DEV - 1)
            load(kt_hbm, kt_v, 3).wait()
            kv16[0] = kt_v[...].astype(jnp.bfloat16)
            for t in range(1, N_DEV):
                kv_copy(0, t).start()
            load(vt_hbm, vt_v, 4).wait()
            kv16[1] = vt_v[...].astype(jnp.bfloat16)
            for t in range(1, N_DEV):
                kv_copy(1, t).start()

        load(x_hbm, x_v, 0).wait()
        load(wq_hbm, wq_v, 1).wait()
        x2 = x_v[...].reshape(B * SQ, D_MODEL)
        q_proj = jnp.dot(x2, wq_v[...],
                         preferred_element_type=jnp.float32)

        row_blk = lax.broadcasted_iota(jnp.int32, (SQ, SKV_LOC), 0) // BLK
        col_blk = lax.broadcasted_iota(jnp.int32, (SQ, SKV_LOC), 1) // BLK
        mask = col_blk <= row_blk

        @pl.when(my != 0)
        def _():
            kv_copy(0, 0).wait_recv()

        weights = []
        for b in range(B):
            for h in range(HQ):
                qh = q_proj[b * SQ:(b + 1) * SQ, h * DH:(h + 1) * DH]
                kh = kv16[0, b, h].astype(jnp.float32)
                s = lax.dot_general(
                    qh, kh, (((1,), (0,)), ((), ())),
                    preferred_element_type=jnp.float32,
                ) * SCALE
                s = jnp.where(mask, s, -1e9)
                m = jnp.max(s, axis=-1, keepdims=True)
                w = jnp.exp(s - m)
                weights.append(w / jnp.sum(w, axis=-1, keepdims=True))

        @pl.when(my != 0)
        def _():
            kv_copy(1, 0).wait_recv()

        ctx_rows = []
        for b in range(B):
            ctx_heads = []
            for h in range(HQ):
                vh = kv16[1, b, h].astype(jnp.float32)
                ctx_heads.append(lax.dot_general(
                    weights[b * HQ + h], vh, (((1,), (1,)), ((), ())),
                    preferred_element_type=jnp.float32,
                ))
            ctx_rows.append(jnp.concatenate(ctx_heads, axis=1))
        ctx = jnp.concatenate(ctx_rows, axis=0)
        load(wo_hbm, wo_v, 2).wait()
        out = jnp.dot(ctx, wo_v[...],
                      preferred_element_type=jnp.float32)
        out_ref[...] = out.reshape(B, SQ, D_MODEL)

        @pl.when(my == 0)
        def _():
            for i in range(2):
                for t in range(1, N_DEV):
                    kv_copy(i, t).wait_send()

    out_shape = jax.ShapeDtypeStruct((B, SQ, D_MODEL), jnp.float32)
    return pl.pallas_call(
        body,
        out_shape=out_shape,
        in_specs=[pl.BlockSpec(memory_space=pltpu.MemorySpace.HBM)] * 5,
        out_specs=pl.BlockSpec(memory_space=pltpu.VMEM),
        scratch_shapes=[
            pltpu.VMEM((B, SQ, D_MODEL), jnp.float32),
            pltpu.VMEM((D_MODEL, HQ * DH), jnp.float32),
            pltpu.VMEM((HQ * DH, D_MODEL), jnp.float32),
            pltpu.VMEM((B, HQ, DH, SKV_LOC), jnp.float32),
            pltpu.VMEM((B, HQ, DH, SKV_LOC), jnp.float32),
            pltpu.VMEM((2, B, HQ, DH, SKV_LOC), jnp.bfloat16),
            pltpu.SemaphoreType.DMA((5,)),
            pltpu.SemaphoreType.DMA((2 * (N_DEV - 1),)),
            pltpu.SemaphoreType.DMA((2,)),
        ],
        compiler_params=pltpu.CompilerParams(collective_id=0),
    )(x, Wq, Kt, Vt, Wo)
